# Initial kernel scaffold; baseline (speedup 1.0000x reference)
import functools

import jax
import jax.numpy as jnp
from jax import lax
from jax.experimental import pallas as pl
from jax.experimental.pallas import tpu as pltpu
from jax.experimental.pallas import tpu_sc as plsc

_NC = 2
_NS = 16
_NW = _NC * _NS
_L = 16


def _tc_tables(x_ref, w_ref, b_ref, s_ref):
    s_ref[...] = (
        jnp.dot(x_ref[...], w_ref[...], preferred_element_type=jnp.float32)
        + b_ref[...]
    )


def _make_sc_score(n_nodes: int, n_edges: int):
    epw = n_edges // _NW

    def _sc_score(s1_hbm, s2_hbm, src_hbm, dst_hbm, out_hbm, t1, t2, si, di, ov):
        wid = lax.axis_index("s") * _NC + lax.axis_index("c")
        base = wid * epw
        pltpu.sync_copy(s1_hbm, t1)
        pltpu.sync_copy(s2_hbm, t2)
        pltpu.sync_copy(src_hbm.at[pl.ds(base, epw)], si)
        pltpu.sync_copy(dst_hbm.at[pl.ds(base, epw)], di)

        def body(i, carry):
            off = i * _L
            ivs = si[pl.ds(off, _L)]
            ivd = di[pl.ds(off, _L)]
            g1 = plsc.load_gather(t1, [ivs])
            g2 = plsc.load_gather(t2, [ivd])
            z = g1 + g2
            ov[pl.ds(off, _L)] = 1.0 / (1.0 + jnp.exp(-z)) + 0.3
            return carry

        lax.fori_loop(0, epw // _L, body, 0)
        pltpu.sync_copy(ov, out_hbm.at[pl.ds(base, epw)])

    sc_call = functools.partial(
        pl.kernel,
        out_type=jax.ShapeDtypeStruct((n_edges,), jnp.float32),
        mesh=plsc.VectorSubcoreMesh(core_axis_name="c", subcore_axis_name="s"),
        scratch_types=[
            pltpu.VMEM((n_nodes,), jnp.float32),
            pltpu.VMEM((n_nodes,), jnp.float32),
            pltpu.VMEM((epw,), jnp.int32),
            pltpu.VMEM((epw,), jnp.int32),
            pltpu.VMEM((epw,), jnp.float32),
        ],
    )(_sc_score)
    return sc_call


def kernel(x, edge_index, W, b):
    n, c = x.shape
    n_edges = edge_index.shape[1]

    wcat = jnp.concatenate([W[:c], W[c:]], axis=1)
    bvec = jnp.concatenate([b, jnp.zeros((1,), b.dtype)]).reshape(1, 2)

    s = pl.pallas_call(
        _tc_tables,
        out_shape=jax.ShapeDtypeStruct((n, 2), jnp.float32),
    )(x, wcat, bvec)

    src = edge_index[0].astype(jnp.int32)
    dst = edge_index[1].astype(jnp.int32)

    e = _make_sc_score(n, n_edges)(s[:, 0], s[:, 1], src, dst)
    return (x, edge_index, e)


# speedup vs baseline: 23.7990x; 23.7990x over previous
import functools

import jax
import jax.numpy as jnp
from jax import lax
from jax.experimental import pallas as pl
from jax.experimental.pallas import tpu as pltpu
from jax.experimental.pallas import tpu_sc as plsc

_NC = 2
_NS = 16
_NW = _NC * _NS
_L = 16


def _tc_tables(x_ref, w_ref, b_ref, s_ref):
    s_ref[...] = (
        jnp.dot(x_ref[...], w_ref[...], preferred_element_type=jnp.float32)
        + b_ref[...]
    )


def _make_sc_score(n_nodes: int, n_edges: int):
    epw = n_edges // _NW

    def _sc_score(s1_hbm, s2_hbm, src_hbm, dst_hbm, out_hbm, t1, t2, si, di, ov):
        wid = lax.axis_index("s") * _NC + lax.axis_index("c")
        base = wid * epw
        pltpu.sync_copy(s1_hbm, t1)
        pltpu.sync_copy(s2_hbm, t2)
        pltpu.sync_copy(src_hbm.at[pl.ds(base, epw)], si)
        pltpu.sync_copy(dst_hbm.at[pl.ds(base, epw)], di)

        def body(i, carry):
            off = i * _L
            ivs = si[pl.ds(off, _L)]
            ivd = di[pl.ds(off, _L)]
            g1 = plsc.load_gather(t1, [ivs])
            g2 = plsc.load_gather(t2, [ivd])
            z = g1 + g2
            ov[pl.ds(off, _L)] = 1.0 / (1.0 + jnp.exp(-z)) + 0.3
            return carry

        lax.fori_loop(0, epw // _L, body, 0)
        pltpu.sync_copy(ov, out_hbm.at[pl.ds(base, epw)])

    sc_call = functools.partial(
        pl.kernel,
        out_type=jax.ShapeDtypeStruct((n_edges,), jnp.float32),
        mesh=plsc.VectorSubcoreMesh(core_axis_name="c", subcore_axis_name="s"),
        compiler_params=pltpu.CompilerParams(needs_layout_passes=False),
        scratch_types=[
            pltpu.VMEM((n_nodes,), jnp.float32),
            pltpu.VMEM((n_nodes,), jnp.float32),
            pltpu.VMEM((epw,), jnp.int32),
            pltpu.VMEM((epw,), jnp.int32),
            pltpu.VMEM((epw,), jnp.float32),
        ],
    )(_sc_score)
    return sc_call


def kernel(x, edge_index, W, b):
    n, c = x.shape
    n_edges = edge_index.shape[1]

    wcat = jnp.concatenate([W[:c], W[c:]], axis=1)
    bvec = jnp.concatenate([b, jnp.zeros((1,), b.dtype)]).reshape(1, 2)

    s = pl.pallas_call(
        _tc_tables,
        out_shape=jax.ShapeDtypeStruct((n, 2), jnp.float32),
    )(x, wcat, bvec)

    src = edge_index[0].astype(jnp.int32)
    dst = edge_index[1].astype(jnp.int32)

    e = _make_sc_score(n, n_edges)(s[:, 0], s[:, 1], src, dst)
    return (x, edge_index, e)


# speedup vs baseline: 26.6791x; 1.1210x over previous
import functools

import jax
import jax.numpy as jnp
from jax import lax
from jax.experimental import pallas as pl
from jax.experimental.pallas import tpu as pltpu
from jax.experimental.pallas import tpu_sc as plsc

_NC = 2
_NS = 16
_NW = _NC * _NS
_L = 16


def _tc_tables(x_ref, w_ref, b_ref, s_ref):
    s_ref[...] = (
        lax.dot_general(
            w_ref[...],
            x_ref[...],
            dimension_numbers=(((1,), (1,)), ((), ())),
            preferred_element_type=jnp.float32,
        )
        + b_ref[...]
    )


def _make_sc_score(n_nodes: int, n_edges: int):
    epw = n_edges // _NW

    def _sc_score(s1_hbm, s2_hbm, src_hbm, dst_hbm, out_hbm, t1, t2, si, di, ov):
        wid = lax.axis_index("s") * _NC + lax.axis_index("c")
        base = wid * epw
        pltpu.sync_copy(s1_hbm, t1)
        pltpu.sync_copy(s2_hbm, t2)
        pltpu.sync_copy(src_hbm.at[pl.ds(base, epw)], si)
        pltpu.sync_copy(dst_hbm.at[pl.ds(base, epw)], di)

        def body(i, carry):
            off = i * _L
            ivs = si[pl.ds(off, _L)]
            ivd = di[pl.ds(off, _L)]
            g1 = plsc.load_gather(t1, [ivs])
            g2 = plsc.load_gather(t2, [ivd])
            z = g1 + g2
            ov[pl.ds(off, _L)] = 1.0 / (1.0 + jnp.exp(-z)) + 0.3
            return carry

        lax.fori_loop(0, epw // _L, body, 0)
        pltpu.sync_copy(ov, out_hbm.at[pl.ds(base, epw)])

    sc_call = functools.partial(
        pl.kernel,
        out_type=jax.ShapeDtypeStruct((n_edges,), jnp.float32),
        mesh=plsc.VectorSubcoreMesh(core_axis_name="c", subcore_axis_name="s"),
        compiler_params=pltpu.CompilerParams(needs_layout_passes=False),
        scratch_types=[
            pltpu.VMEM((n_nodes,), jnp.float32),
            pltpu.VMEM((n_nodes,), jnp.float32),
            pltpu.VMEM((epw,), jnp.int32),
            pltpu.VMEM((epw,), jnp.int32),
            pltpu.VMEM((epw,), jnp.float32),
        ],
    )(_sc_score)
    return sc_call


def kernel(x, edge_index, W, b):
    n, c = x.shape
    n_edges = edge_index.shape[1]

    wcat = jnp.concatenate([W[:c], W[c:]], axis=1).T
    bvec = jnp.concatenate([b, jnp.zeros((1,), b.dtype)]).reshape(2, 1)

    s = pl.pallas_call(
        _tc_tables,
        out_shape=jax.ShapeDtypeStruct((2, n), jnp.float32),
    )(x, wcat, bvec)

    ei = edge_index.astype(jnp.int32)
    e = _make_sc_score(n, n_edges)(s[0], s[1], ei[0], ei[1])
    return (x, edge_index, e)


# speedup vs baseline: 32.9199x; 1.2339x over previous
import functools

import jax
import jax.numpy as jnp
from jax import lax
from jax.experimental import pallas as pl
from jax.experimental.pallas import tpu as pltpu
from jax.experimental.pallas import tpu_sc as plsc

_NC = 2
_NS = 16
_NW = _NC * _NS
_L = 16


def _tc_tables(x_ref, w_ref, b_ref, s_ref):
    s_ref[...] = (
        lax.dot_general(
            w_ref[...],
            x_ref[...],
            dimension_numbers=(((1,), (1,)), ((), ())),
            preferred_element_type=jnp.float32,
        )
        + b_ref[...]
    )


def _make_sc_score(n_nodes: int, n_edges: int):
    epw = n_edges // _NW

    def _sc_score(s1_hbm, s2_hbm, src_hbm, dst_hbm, out_hbm, t1, t2, si, di, ov):
        wid = lax.axis_index("s") * _NC + lax.axis_index("c")
        base = wid * epw
        pltpu.sync_copy(s1_hbm, t1)
        pltpu.sync_copy(s2_hbm, t2)
        pltpu.sync_copy(src_hbm.at[pl.ds(base, epw)], si)
        pltpu.sync_copy(dst_hbm.at[pl.ds(base, epw)], di)

        @plsc.parallel_loop(0, epw, step=_L, unroll=8)
        def _body(off):
            ivs = si[pl.ds(off, _L)]
            ivd = di[pl.ds(off, _L)]
            g1 = plsc.load_gather(t1, [ivs])
            g2 = plsc.load_gather(t2, [ivd])
            z = g1 + g2
            ov[pl.ds(off, _L)] = 1.0 / (1.0 + jnp.exp(-z)) + 0.3
        pltpu.sync_copy(ov, out_hbm.at[pl.ds(base, epw)])

    sc_call = functools.partial(
        pl.kernel,
        out_type=jax.ShapeDtypeStruct((n_edges,), jnp.float32),
        mesh=plsc.VectorSubcoreMesh(core_axis_name="c", subcore_axis_name="s"),
        compiler_params=pltpu.CompilerParams(needs_layout_passes=False),
        scratch_types=[
            pltpu.VMEM((n_nodes,), jnp.float32),
            pltpu.VMEM((n_nodes,), jnp.float32),
            pltpu.VMEM((epw,), jnp.int32),
            pltpu.VMEM((epw,), jnp.int32),
            pltpu.VMEM((epw,), jnp.float32),
        ],
    )(_sc_score)
    return sc_call


def kernel(x, edge_index, W, b):
    n, c = x.shape
    n_edges = edge_index.shape[1]

    wcat = jnp.concatenate([W[:c], W[c:]], axis=1).T
    bvec = jnp.concatenate([b, jnp.zeros((1,), b.dtype)]).reshape(2, 1)

    s = pl.pallas_call(
        _tc_tables,
        out_shape=jax.ShapeDtypeStruct((2, n), jnp.float32),
    )(x, wcat, bvec)

    ei = edge_index.astype(jnp.int32)
    e = _make_sc_score(n, n_edges)(s[0], s[1], ei[0], ei[1])
    return (x, edge_index, e)


# speedup vs baseline: 43.6254x; 1.3252x over previous
import functools

import jax
import jax.numpy as jnp
from jax import lax
from jax.experimental import pallas as pl
from jax.experimental.pallas import tpu as pltpu
from jax.experimental.pallas import tpu_sc as plsc

_NC = 2
_NS = 16
_NW = _NC * _NS
_L = 16


def _tc_tables(x_ref, w_ref, b_ref, s_ref):
    s_ref[...] = (
        lax.dot_general(
            w_ref[...],
            x_ref[...],
            dimension_numbers=(((1,), (1,)), ((), ())),
            preferred_element_type=jnp.float32,
        )
        + b_ref[...]
    )


_BLK = 128


def _make_sc_score(n_nodes: int, n_edges: int):
    nblk = n_edges // _BLK
    lo_blk, rem = divmod(nblk, _NW)
    hi_cnt = (lo_blk + 1) * _BLK

    def _sc_score(s1_hbm, s2_hbm, edge_hbm, out_hbm, t1, t2, exy, ov):
        wid = lax.axis_index("s") * _NC + lax.axis_index("c")
        base = (wid * lo_blk + jnp.minimum(wid, rem)) * _BLK
        pltpu.sync_copy(s1_hbm, t1)
        pltpu.sync_copy(s2_hbm, t2)

        def work(cnt):
            pltpu.sync_copy(
                edge_hbm.at[:, pl.ds(base, cnt)], exy.at[:, pl.ds(0, cnt)]
            )

            @plsc.parallel_loop(0, cnt, step=_L, unroll=8)
            def _body(off):
                ivs = exy[0, pl.ds(off, _L)]
                ivd = exy[1, pl.ds(off, _L)]
                g1 = plsc.load_gather(t1, [ivs])
                g2 = plsc.load_gather(t2, [ivd])
                z = g1 + g2
                ov[pl.ds(off, _L)] = 1.0 / (1.0 + jnp.exp(-z)) + 0.3

            pltpu.sync_copy(ov.at[pl.ds(0, cnt)], out_hbm.at[pl.ds(base, cnt)])

        pl.when(wid < rem)(lambda: work((lo_blk + 1) * _BLK))
        pl.when(wid >= rem)(lambda: work(lo_blk * _BLK))

    sc_call = functools.partial(
        pl.kernel,
        out_type=jax.ShapeDtypeStruct((n_edges,), jnp.float32),
        mesh=plsc.VectorSubcoreMesh(core_axis_name="c", subcore_axis_name="s"),
        compiler_params=pltpu.CompilerParams(needs_layout_passes=False),
        scratch_types=[
            pltpu.VMEM((n_nodes,), jnp.float32),
            pltpu.VMEM((n_nodes,), jnp.float32),
            pltpu.VMEM((2, hi_cnt), jnp.int32),
            pltpu.VMEM((hi_cnt,), jnp.float32),
        ],
    )(_sc_score)
    return sc_call


def kernel(x, edge_index, W, b):
    n, c = x.shape
    n_edges = edge_index.shape[1]

    wcat = jnp.concatenate([W[:c], W[c:]], axis=1).T
    bvec = jnp.concatenate([b, jnp.zeros((1,), b.dtype)]).reshape(2, 1)

    s = pl.pallas_call(
        _tc_tables,
        out_shape=jax.ShapeDtypeStruct((2, n), jnp.float32),
    )(x, wcat, bvec)

    e = _make_sc_score(n, n_edges)(s[0], s[1], edge_index.astype(jnp.int32))
    return (x, edge_index, e)
